# baseline (speedup 1.0000x reference)
import jax
import jax.numpy as jnp
from jax import lax
from jax.experimental import pallas as pl
from jax.experimental.pallas import tpu as pltpu
from jax.experimental.pallas import tpu_sc as plsc

NC, NS = 2, 16
NW = NC * NS
CHUNKS = (56, 56, 56, 56, 32)
BUF_ROWS = max(CHUNKS)


def _sc_body(table_hbm, out_hbm, buf):
    wid = lax.axis_index("s") * NC + lax.axis_index("c")
    rows_per_w = table_hbm.shape[0] // NW
    base = wid * rows_per_w

    off = 0
    for c in CHUNKS:
        row = base + off
        pltpu.sync_copy(table_hbm.at[pl.ds(row, c)], buf.at[pl.ds(0, c)])
        for b in range(4):
            pltpu.sync_copy(buf.at[pl.ds(0, c)], out_hbm.at[b, pl.ds(row, c)])
        off += c


def kernel(B, T, pos_weight):
    t_static, d = pos_weight.shape
    run = pl.kernel(
        _sc_body,
        out_type=jax.ShapeDtypeStruct((4, t_static, d), pos_weight.dtype),
        mesh=plsc.VectorSubcoreMesh(core_axis_name="c", subcore_axis_name="s"),
        scratch_types=[
            pltpu.VMEM((BUF_ROWS, d), jnp.float32),
        ],
    )
    return run(pos_weight)


# speedup vs baseline: 1.0663x; 1.0663x over previous
import jax
import jax.numpy as jnp
from jax import lax
from jax.experimental import pallas as pl
from jax.experimental.pallas import tpu as pltpu
from jax.experimental.pallas import tpu_sc as plsc

NC, NS = 2, 16
NW = NC * NS
SC_CHUNK = 48
T_SPLIT = 5120

T_BLK = 1024
NBUF = 5


def _sc_body(table_hbm, out_hbm, buf):
    wid = lax.axis_index("s") * NC + lax.axis_index("c")
    rows_per_w = (table_hbm.shape[0] - T_SPLIT) // NW
    base = T_SPLIT + wid * rows_per_w

    def step(i, carry):
        row = base + i * SC_CHUNK
        pltpu.sync_copy(table_hbm.at[pl.ds(row, SC_CHUNK)], buf)
        for b in range(4):
            pltpu.sync_copy(buf, out_hbm.at[b, pl.ds(row, SC_CHUNK)])
        return carry

    lax.fori_loop(0, rows_per_w // SC_CHUNK, step, 0)


def _tc_body(w_hbm, prev_hbm, o_hbm, buf, rsem, wsem):
    n = T_SPLIT // T_BLK

    def rd(i, s):
        return pltpu.make_async_copy(
            w_hbm.at[pl.ds(i * T_BLK, T_BLK)], buf.at[s], rsem.at[s]
        )

    def wr(b, i, s):
        return pltpu.make_async_copy(
            buf.at[s], o_hbm.at[b, pl.ds(i * T_BLK, T_BLK)], wsem.at[s]
        )

    for s in range(NBUF):
        rd(s, s).start()

    def step(g, carry):
        for s in range(NBUF):
            i = g * NBUF + s
            rd(i, s).wait()
            for b in range(4):
                wr(b, i, s).start()
        for s in range(NBUF):
            i = g * NBUF + s
            for b in range(4):
                wr(b, i, s).wait()
            nxt = i + NBUF

            @pl.when(nxt < n)
            def _():
                rd(nxt, s).start()

        return carry

    lax.fori_loop(0, n // NBUF, step, 0)


def kernel(B, T, pos_weight):
    t_static, d = pos_weight.shape

    sc_run = pl.kernel(
        _sc_body,
        out_type=jax.ShapeDtypeStruct((4, t_static, d), pos_weight.dtype),
        mesh=plsc.VectorSubcoreMesh(core_axis_name="c", subcore_axis_name="s"),
        scratch_types=[
            pltpu.VMEM((SC_CHUNK, d), jnp.float32),
        ],
    )
    partial = sc_run(pos_weight)

    out = pl.pallas_call(
        _tc_body,
        in_specs=[
            pl.BlockSpec(memory_space=pltpu.MemorySpace.HBM),
            pl.BlockSpec(memory_space=pltpu.MemorySpace.HBM),
        ],
        out_specs=pl.BlockSpec(memory_space=pltpu.MemorySpace.HBM),
        out_shape=jax.ShapeDtypeStruct((4, t_static, d), pos_weight.dtype),
        input_output_aliases={1: 0},
        scratch_shapes=[
            pltpu.VMEM((NBUF, T_BLK, d), jnp.float32),
            pltpu.SemaphoreType.DMA((NBUF,)),
            pltpu.SemaphoreType.DMA((NBUF,)),
        ],
    )(pos_weight, partial)
    return out


# speedup vs baseline: 1.0713x; 1.0047x over previous
import jax
import jax.numpy as jnp
from jax import lax
from jax.experimental import pallas as pl
from jax.experimental.pallas import tpu as pltpu
from jax.experimental.pallas import tpu_sc as plsc

NC, NS = 2, 16
NW = NC * NS
SC_CHUNK = 48
T_SPLIT = 5120

T_BLK = 2560
NBUF = 2


def _sc_body(table_hbm, out_hbm, buf):
    wid = lax.axis_index("s") * NC + lax.axis_index("c")
    rows_per_w = (table_hbm.shape[0] - T_SPLIT) // NW
    base = T_SPLIT + wid * rows_per_w

    def step(i, carry):
        row = base + i * SC_CHUNK
        pltpu.sync_copy(table_hbm.at[pl.ds(row, SC_CHUNK)], buf)
        for b in range(4):
            pltpu.sync_copy(buf, out_hbm.at[b, pl.ds(row, SC_CHUNK)])
        return carry

    lax.fori_loop(0, rows_per_w // SC_CHUNK, step, 0)


def _tc_body(w_hbm, prev_hbm, o_hbm, buf, rsem, wsem):
    n = T_SPLIT // T_BLK

    def rd(i, s):
        return pltpu.make_async_copy(
            w_hbm.at[pl.ds(i * T_BLK, T_BLK)], buf.at[s], rsem.at[s]
        )

    def wr(b, i, s):
        return pltpu.make_async_copy(
            buf.at[s], o_hbm.at[b, pl.ds(i * T_BLK, T_BLK)], wsem.at[s]
        )

    for s in range(NBUF):
        rd(s, s).start()

    def step(g, carry):
        for s in range(NBUF):
            i = g * NBUF + s
            rd(i, s).wait()
            for b in range(4):
                wr(b, i, s).start()
        for s in range(NBUF):
            i = g * NBUF + s
            for b in range(4):
                wr(b, i, s).wait()
            nxt = i + NBUF

            @pl.when(nxt < n)
            def _():
                rd(nxt, s).start()

        return carry

    lax.fori_loop(0, n // NBUF, step, 0)


def kernel(B, T, pos_weight):
    t_static, d = pos_weight.shape

    sc_run = pl.kernel(
        _sc_body,
        out_type=jax.ShapeDtypeStruct((4, t_static, d), pos_weight.dtype),
        mesh=plsc.VectorSubcoreMesh(core_axis_name="c", subcore_axis_name="s"),
        scratch_types=[
            pltpu.VMEM((SC_CHUNK, d), jnp.float32),
        ],
    )
    partial = sc_run(pos_weight)

    out = pl.pallas_call(
        _tc_body,
        in_specs=[
            pl.BlockSpec(memory_space=pltpu.MemorySpace.HBM),
            pl.BlockSpec(memory_space=pltpu.MemorySpace.HBM),
        ],
        out_specs=pl.BlockSpec(memory_space=pltpu.MemorySpace.HBM),
        out_shape=jax.ShapeDtypeStruct((4, t_static, d), pos_weight.dtype),
        input_output_aliases={1: 0},
        scratch_shapes=[
            pltpu.VMEM((NBUF, T_BLK, d), jnp.float32),
            pltpu.SemaphoreType.DMA((NBUF,)),
            pltpu.SemaphoreType.DMA((NBUF,)),
        ],
    )(pos_weight, partial)
    return out
